# Initial kernel scaffold; baseline (speedup 1.0000x reference)
import jax
import jax.numpy as jnp
from jax.experimental import pallas as pl


def kernel(x, adj, poly_item, W1, b1, W2, b2, filter_param):
    raise NotImplementedError("write your pallas kernel here")


# speedup vs baseline: 1.8596x; 1.8596x over previous
import functools
import math

import jax
import jax.numpy as jnp
from jax.experimental import pallas as pl

_K = 5


def _mlp_body(x_ref, W1_ref, b1_ref, W2_ref, b2_ref, c_ref, h_ref, acc_ref):
    h = jnp.dot(x_ref[...], W1_ref[...], preferred_element_type=jnp.float32)
    h = jnp.maximum(h + b1_ref[...], 0.0)
    h = jnp.dot(h, W2_ref[...], preferred_element_type=jnp.float32) + b2_ref[...]
    h_ref[...] = h
    acc_ref[...] = c_ref[0, 0] * h


def _step_body(adj_ref, poly_ref, p_ref, acc_ref, c_ref, pnew_ref, accnew_ref,
               *, last):
    pnew = jnp.dot(adj_ref[...], p_ref[...], preferred_element_type=jnp.float32)
    accnew = (jnp.dot(poly_ref[...], acc_ref[...],
                      preferred_element_type=jnp.float32)
              + c_ref[0, 0] * pnew)
    pnew_ref[...] = pnew
    if last:
        m = jnp.max(accnew, axis=1, keepdims=True)
        lse = jnp.log(jnp.sum(jnp.exp(accnew - m), axis=1, keepdims=True)) + m
        accnew_ref[...] = accnew - lse
    else:
        accnew_ref[...] = accnew


def kernel(x, adj, poly_item, W1, b1, W2, b2, filter_param):
    N, D_IN = x.shape
    D_HID = W1.shape[1]
    D_OUT = W2.shape[1]

    fp = jax.nn.relu(filter_param[:, 0])
    binom = jnp.asarray([math.comb(_K, i) / 2.0 ** _K for i in range(_K + 1)],
                        jnp.float32)
    coefs = (binom * fp).reshape(_K + 1, 1, 1)

    BM = 256
    grid = (N // BM,)

    h, acc = pl.pallas_call(
        _mlp_body,
        grid=grid,
        in_specs=[
            pl.BlockSpec((BM, D_IN), lambda i: (i, 0)),
            pl.BlockSpec((D_IN, D_HID), lambda i: (0, 0)),
            pl.BlockSpec((1, D_HID), lambda i: (0, 0)),
            pl.BlockSpec((D_HID, D_OUT), lambda i: (0, 0)),
            pl.BlockSpec((1, D_OUT), lambda i: (0, 0)),
            pl.BlockSpec((1, 1), lambda i: (0, 0)),
        ],
        out_specs=[pl.BlockSpec((BM, D_OUT), lambda i: (i, 0))] * 2,
        out_shape=[jax.ShapeDtypeStruct((N, D_OUT), jnp.float32)] * 2,
    )(x, W1, b1.reshape(1, -1), W2, b2.reshape(1, -1), coefs[_K])

    p = h
    for t in range(1, _K + 1):
        p, acc = pl.pallas_call(
            functools.partial(_step_body, last=(t == _K)),
            grid=grid,
            in_specs=[
                pl.BlockSpec((BM, N), lambda i: (i, 0)),
                pl.BlockSpec((BM, N), lambda i: (i, 0)),
                pl.BlockSpec((N, D_OUT), lambda i: (0, 0)),
                pl.BlockSpec((N, D_OUT), lambda i: (0, 0)),
                pl.BlockSpec((1, 1), lambda i: (0, 0)),
            ],
            out_specs=[pl.BlockSpec((BM, D_OUT), lambda i: (i, 0))] * 2,
            out_shape=[jax.ShapeDtypeStruct((N, D_OUT), jnp.float32)] * 2,
        )(adj, poly_item, p, acc, coefs[_K - t])
    return acc


# speedup vs baseline: 2.0880x; 1.1229x over previous
import functools
import math

import jax
import jax.numpy as jnp
from jax.experimental import pallas as pl

_K = 5


def _mlp_body(x_ref, W1_ref, b1_ref, W2_ref, b2_ref, c_ref, h_ref, acc_ref):
    h = jnp.dot(x_ref[...], W1_ref[...], preferred_element_type=jnp.float32)
    h = jnp.maximum(h + b1_ref[...], 0.0)
    h = jnp.dot(h, W2_ref[...], preferred_element_type=jnp.float32) + b2_ref[...]
    h_ref[...] = h
    acc_ref[...] = c_ref[0, 0] * h


def _step1_body(adj_ref, poly_ref, p_ref, acc_ref, c_ref,
                pnew_ref, accnew_ref, adj16_ref, poly16_ref):
    a = adj_ref[...]
    q = poly_ref[...]
    adj16_ref[...] = a.astype(jnp.bfloat16)
    poly16_ref[...] = q.astype(jnp.bfloat16)
    pnew = jnp.dot(a, p_ref[...], preferred_element_type=jnp.float32)
    accnew = (jnp.dot(q, acc_ref[...], preferred_element_type=jnp.float32)
              + c_ref[0, 0] * pnew)
    pnew_ref[...] = pnew.astype(jnp.bfloat16)
    accnew_ref[...] = accnew


def _step_body(adj_ref, poly_ref, p_ref, acc_ref, c_ref, pnew_ref, accnew_ref,
               *, last):
    pnew = jnp.dot(adj_ref[...], p_ref[...], preferred_element_type=jnp.float32)
    accnew = (jnp.dot(poly_ref[...], acc_ref[...].astype(jnp.bfloat16),
                      preferred_element_type=jnp.float32)
              + c_ref[0, 0] * pnew)
    if not last:
        pnew_ref[...] = pnew.astype(jnp.bfloat16)
        accnew_ref[...] = accnew
    else:
        pnew_ref[...] = pnew.astype(jnp.bfloat16)
        m = jnp.max(accnew, axis=1, keepdims=True)
        lse = jnp.log(jnp.sum(jnp.exp(accnew - m), axis=1, keepdims=True)) + m
        accnew_ref[...] = accnew - lse


def kernel(x, adj, poly_item, W1, b1, W2, b2, filter_param):
    N, D_IN = x.shape
    D_HID = W1.shape[1]
    D_OUT = W2.shape[1]

    fp = jax.nn.relu(filter_param[:, 0])
    binom = jnp.asarray([math.comb(_K, i) / 2.0 ** _K for i in range(_K + 1)],
                        jnp.float32)
    coefs = (binom * fp).reshape(_K + 1, 1, 1)

    BM = 256
    grid = (N // BM,)
    row_out = lambda i: (i, 0)
    whole = lambda i: (0, 0)

    h, acc = pl.pallas_call(
        _mlp_body,
        grid=grid,
        in_specs=[
            pl.BlockSpec((BM, D_IN), row_out),
            pl.BlockSpec((D_IN, D_HID), whole),
            pl.BlockSpec((1, D_HID), whole),
            pl.BlockSpec((D_HID, D_OUT), whole),
            pl.BlockSpec((1, D_OUT), whole),
            pl.BlockSpec((1, 1), whole),
        ],
        out_specs=[pl.BlockSpec((BM, D_OUT), row_out)] * 2,
        out_shape=[jax.ShapeDtypeStruct((N, D_OUT), jnp.float32)] * 2,
    )(x, W1, b1.reshape(1, -1), W2, b2.reshape(1, -1), coefs[_K])

    p, acc, adj16, poly16 = pl.pallas_call(
        _step1_body,
        grid=grid,
        in_specs=[
            pl.BlockSpec((BM, N), row_out),
            pl.BlockSpec((BM, N), row_out),
            pl.BlockSpec((N, D_OUT), whole),
            pl.BlockSpec((N, D_OUT), whole),
            pl.BlockSpec((1, 1), whole),
        ],
        out_specs=[
            pl.BlockSpec((BM, D_OUT), row_out),
            pl.BlockSpec((BM, D_OUT), row_out),
            pl.BlockSpec((BM, N), row_out),
            pl.BlockSpec((BM, N), row_out),
        ],
        out_shape=[
            jax.ShapeDtypeStruct((N, D_OUT), jnp.bfloat16),
            jax.ShapeDtypeStruct((N, D_OUT), jnp.float32),
            jax.ShapeDtypeStruct((N, N), jnp.bfloat16),
            jax.ShapeDtypeStruct((N, N), jnp.bfloat16),
        ],
    )(adj, poly_item, h, acc, coefs[_K - 1])

    for t in range(2, _K + 1):
        p, acc = pl.pallas_call(
            functools.partial(_step_body, last=(t == _K)),
            grid=grid,
            in_specs=[
                pl.BlockSpec((BM, N), row_out),
                pl.BlockSpec((BM, N), row_out),
                pl.BlockSpec((N, D_OUT), whole),
                pl.BlockSpec((N, D_OUT), whole),
                pl.BlockSpec((1, 1), whole),
            ],
            out_specs=[pl.BlockSpec((BM, D_OUT), row_out)] * 2,
            out_shape=[
                jax.ShapeDtypeStruct((N, D_OUT), jnp.bfloat16),
                jax.ShapeDtypeStruct((N, D_OUT), jnp.float32),
            ],
        )(adj16, poly16, p, acc, coefs[_K - t])
    return acc


# speedup vs baseline: 2.3947x; 1.1469x over previous
import functools
import math

import jax
import jax.numpy as jnp
from jax import lax
from jax.experimental import pallas as pl
from jax.experimental.pallas import tpu as pltpu

_K = 5
_BR = 128


def _mlp_body(x_ref, W1_ref, b1_ref, W2_ref, b2_ref, h_ref):
    h = jnp.dot(x_ref[...], W1_ref[...], preferred_element_type=jnp.float32)
    h = jnp.maximum(h + b1_ref[...], 0.0)
    h_ref[...] = (jnp.dot(h, W2_ref[...], preferred_element_type=jnp.float32)
                  + b2_ref[...])


def _mega_body(adj_hbm, poly_hbm, h_ref, c_ref,
               y_ref, p16_hbm,
               a16, fa, fp, pstage, pbuf, acc16,
               sa, sp, so, si):
    n = adj_hbm.shape[0]
    nb = n // _BR

    def a_in(b, slot):
        return pltpu.make_async_copy(
            adj_hbm.at[pl.ds(b * _BR, _BR), :], fa.at[slot], sa.at[slot])

    def p_in(b, slot):
        return pltpu.make_async_copy(
            poly_hbm.at[pl.ds(b * _BR, _BR), :], fp.at[slot], sp.at[slot])

    def p16_out(b, slot):
        return pltpu.make_async_copy(
            pstage.at[slot], p16_hbm.at[pl.ds(b * _BR, _BR), :], so.at[slot])

    def p16_in(b, slot):
        return pltpu.make_async_copy(
            p16_hbm.at[pl.ds(b * _BR, _BR), :], pstage.at[slot], si.at[slot])

    h = h_ref[...]
    pbuf[0, :, :] = h.astype(jnp.bfloat16)
    acc16[0, :, :] = (c_ref[_K, 0] * h).astype(jnp.bfloat16)

    a_in(0, 0).start()
    p_in(0, 0).start()
    a_in(1, 1).start()
    p_in(1, 1).start()

    def body1(b, _):
        slot = lax.rem(b, 2)
        rows = pl.ds(b * _BR, _BR)
        a_in(b, slot).wait()
        p_in(b, slot).wait()
        ablk = fa[slot].astype(jnp.bfloat16)
        a16[rows, :] = ablk
        pblk = fp[slot].astype(jnp.bfloat16)

        @pl.when(b >= 2)
        def _wait_out():
            p16_out(b - 2, slot).wait()

        pstage[slot, :, :] = pblk
        p16_out(b, slot).start()

        @pl.when(b + 2 < nb)
        def _next():
            a_in(b + 2, slot).start()
            p_in(b + 2, slot).start()

        pnew = jnp.dot(ablk, pbuf[0], preferred_element_type=jnp.float32)
        accn = (jnp.dot(pblk, acc16[0], preferred_element_type=jnp.float32)
                + c_ref[_K - 1, 0] * pnew)
        pbuf[1, rows, :] = pnew.astype(jnp.bfloat16)
        acc16[1, rows, :] = accn.astype(jnp.bfloat16)
        return 0

    lax.fori_loop(0, nb, body1, 0)
    p16_out(nb - 2, 0).wait()
    p16_out(nb - 1, 1).wait()

    for t in range(2, _K + 1):
        cur = (t - 1) % 2
        nxt = t % 2
        last = t == _K
        p16_in(0, 0).start()
        p16_in(1, 1).start()

        def bodyt(b, _, cur=cur, nxt=nxt, last=last, t=t):
            slot = lax.rem(b, 2)
            rows = pl.ds(b * _BR, _BR)
            p16_in(b, slot).wait()
            pblk = pstage[slot]

            @pl.when(b + 2 < nb)
            def _next():
                p16_in(b + 2, slot).start()

            pnew = jnp.dot(a16[rows, :], pbuf[cur],
                           preferred_element_type=jnp.float32)
            accn = (jnp.dot(pblk, acc16[cur], preferred_element_type=jnp.float32)
                    + c_ref[_K - t, 0] * pnew)
            if not last:
                pbuf[nxt, rows, :] = pnew.astype(jnp.bfloat16)
                acc16[nxt, rows, :] = accn.astype(jnp.bfloat16)
            else:
                m = jnp.max(accn, axis=1, keepdims=True)
                lse = (jnp.log(jnp.sum(jnp.exp(accn - m), axis=1,
                                       keepdims=True)) + m)
                y_ref[rows, :] = accn - lse
            return 0

        lax.fori_loop(0, nb, bodyt, 0)


def kernel(x, adj, poly_item, W1, b1, W2, b2, filter_param):
    N, D_IN = x.shape
    D_HID = W1.shape[1]
    D_OUT = W2.shape[1]

    fp = jax.nn.relu(filter_param[:, 0])
    binom = jnp.asarray([math.comb(_K, i) / 2.0 ** _K for i in range(_K + 1)],
                        jnp.float32)
    coefs = jnp.zeros((8, 1), jnp.float32).at[:_K + 1, 0].set(binom * fp)

    BM = 256
    h = pl.pallas_call(
        _mlp_body,
        grid=(N // BM,),
        in_specs=[
            pl.BlockSpec((BM, D_IN), lambda i: (i, 0)),
            pl.BlockSpec((D_IN, D_HID), lambda i: (0, 0)),
            pl.BlockSpec((1, D_HID), lambda i: (0, 0)),
            pl.BlockSpec((D_HID, D_OUT), lambda i: (0, 0)),
            pl.BlockSpec((1, D_OUT), lambda i: (0, 0)),
        ],
        out_specs=pl.BlockSpec((BM, D_OUT), lambda i: (i, 0)),
        out_shape=jax.ShapeDtypeStruct((N, D_OUT), jnp.float32),
    )(x, W1, b1.reshape(1, -1), W2, b2.reshape(1, -1))

    y, _ = pl.pallas_call(
        _mega_body,
        in_specs=[
            pl.BlockSpec(memory_space=pl.ANY),
            pl.BlockSpec(memory_space=pl.ANY),
            pl.BlockSpec(memory_space=pltpu.VMEM),
            pl.BlockSpec(memory_space=pltpu.SMEM),
        ],
        out_specs=[
            pl.BlockSpec(memory_space=pltpu.VMEM),
            pl.BlockSpec(memory_space=pl.ANY),
        ],
        out_shape=[
            jax.ShapeDtypeStruct((N, D_OUT), jnp.float32),
            jax.ShapeDtypeStruct((N, N), jnp.bfloat16),
        ],
        scratch_shapes=[
            pltpu.VMEM((N, N), jnp.bfloat16),
            pltpu.VMEM((2, _BR, N), jnp.float32),
            pltpu.VMEM((2, _BR, N), jnp.float32),
            pltpu.VMEM((2, _BR, N), jnp.bfloat16),
            pltpu.VMEM((2, N, D_OUT), jnp.bfloat16),
            pltpu.VMEM((2, N, D_OUT), jnp.bfloat16),
            pltpu.SemaphoreType.DMA((2,)),
            pltpu.SemaphoreType.DMA((2,)),
            pltpu.SemaphoreType.DMA((2,)),
            pltpu.SemaphoreType.DMA((2,)),
        ],
        compiler_params=pltpu.CompilerParams(
            vmem_limit_bytes=100 * 1024 * 1024),
    )(adj, poly_item, h, coefs)
    return y


# speedup vs baseline: 2.8194x; 1.1774x over previous
import math

import jax
import jax.numpy as jnp
from jax import lax
from jax.experimental import pallas as pl
from jax.experimental.pallas import tpu as pltpu

_K = 5
_BR = 128
_BC = 512
_NPR = 0


def _mlp_body(x_ref, W1_ref, b1_ref, W2_ref, b2_ref, h_ref):
    h = jnp.dot(x_ref[...], W1_ref[...], preferred_element_type=jnp.float32)
    h = jnp.maximum(h + b1_ref[...], 0.0)
    h_ref[...] = (jnp.dot(h, W2_ref[...], preferred_element_type=jnp.float32)
                  + b2_ref[...]).astype(jnp.bfloat16)


def _mega_body(adj_hbm, poly_hbm, h_ref, c_ref,
               y_ref, p16_hbm,
               a16, p16r, fa, fp, pstage, pbuf, acc16,
               sa, sp, so, si):
    n = adj_hbm.shape[0]
    nb = n // _BR
    nres = _NPR * _BC // _BR

    def a_in(b, slot):
        return pltpu.make_async_copy(
            adj_hbm.at[pl.ds(b * _BR, _BR), :], fa.at[slot], sa.at[slot])

    def p_in(b, slot):
        return pltpu.make_async_copy(
            poly_hbm.at[pl.ds(b * _BR, _BR), :], fp.at[slot], sp.at[slot])

    def p16_out(b, slot):
        return pltpu.make_async_copy(
            pstage.at[slot, pl.ds(0, _BR)],
            p16_hbm.at[pl.ds(b * _BR, _BR), :], so.at[slot])

    def p16_in(b, slot):
        return pltpu.make_async_copy(
            p16_hbm.at[pl.ds(b * _BC, _BC), :], pstage.at[slot], si.at[slot])

    h16 = h_ref[...]
    pbuf[0, :, :] = h16
    acc16[0, :, :] = (c_ref[_K, 0] * h16.astype(jnp.float32)).astype(jnp.bfloat16)

    a_in(0, 0).start()
    p_in(0, 0).start()
    a_in(1, 1).start()
    p_in(1, 1).start()

    def body1(b, _):
        slot = lax.rem(b, 2)
        rows = pl.ds(b * _BR, _BR)
        a_in(b, slot).wait()
        p_in(b, slot).wait()
        ablk = fa[slot].astype(jnp.bfloat16)
        pblk = fp[slot].astype(jnp.bfloat16)
        a16[rows, :] = ablk

        @pl.when(b < nres)
        def _store_res():
            p16r[rows, :] = pblk

        @pl.when(b >= nres + 2)
        def _wait_out():
            p16_out(b - 2, slot).wait()

        @pl.when(b >= nres)
        def _stage_out():
            pstage[slot, pl.ds(0, _BR), :] = pblk
            p16_out(b, slot).start()

        @pl.when(b + 2 < nb)
        def _next():
            a_in(b + 2, slot).start()
            p_in(b + 2, slot).start()

        pnew = jnp.dot(ablk, pbuf[0], preferred_element_type=jnp.float32)
        accn = (jnp.dot(pblk, acc16[0], preferred_element_type=jnp.float32)
                + c_ref[_K - 1, 0] * pnew)
        pbuf[1, rows, :] = pnew.astype(jnp.bfloat16)
        acc16[1, rows, :] = accn.astype(jnp.bfloat16)
        return 0

    lax.fori_loop(0, nb, body1, 0)
    p16_out(nb - 2, 0).wait()
    p16_out(nb - 1, 1).wait()

    nbc = n // _BC
    for t in range(2, _K + 1):
        cur = (t - 1) % 2
        nxt = t % 2
        last = t == _K
        p16_in(_NPR, 0).start()
        if _NPR + 1 < nbc:
            p16_in(_NPR + 1, 1).start()

        for b in range(nbc):
            rows = pl.ds(b * _BC, _BC)
            if b < _NPR:
                pblk_ref = p16r.at[pl.ds(b * _BC, _BC), :]
            else:
                slot = b % 2
                p16_in(b, slot).wait()
                pblk_ref = pstage.at[slot]
            pnew = jnp.dot(a16[rows, :], pbuf[cur],
                           preferred_element_type=jnp.float32)
            accn = (jnp.dot(pblk_ref[...], acc16[cur],
                            preferred_element_type=jnp.float32)
                    + c_ref[_K - t, 0] * pnew)
            if b + 2 < nbc and b + 2 >= _NPR + 2:
                p16_in(b + 2, (b + 2) % 2).start()
            if not last:
                pbuf[nxt, rows, :] = pnew.astype(jnp.bfloat16)
                acc16[nxt, rows, :] = accn.astype(jnp.bfloat16)
            else:
                m = jnp.max(accn, axis=1, keepdims=True)
                lse = (jnp.log(jnp.sum(jnp.exp(accn - m), axis=1,
                                       keepdims=True)) + m)
                y_ref[rows, :] = accn - lse


def kernel(x, adj, poly_item, W1, b1, W2, b2, filter_param):
    N, D_IN = x.shape
    D_HID = W1.shape[1]
    D_OUT = W2.shape[1]

    fp = jax.nn.relu(filter_param[:, 0])
    binom = jnp.asarray([math.comb(_K, i) / 2.0 ** _K for i in range(_K + 1)],
                        jnp.float32)
    coefs = jnp.zeros((8, 1), jnp.float32).at[:_K + 1, 0].set(binom * fp)

    BM = 256
    h = pl.pallas_call(
        _mlp_body,
        grid=(N // BM,),
        in_specs=[
            pl.BlockSpec((BM, D_IN), lambda i: (i, 0)),
            pl.BlockSpec((D_IN, D_HID), lambda i: (0, 0)),
            pl.BlockSpec((1, D_HID), lambda i: (0, 0)),
            pl.BlockSpec((D_HID, D_OUT), lambda i: (0, 0)),
            pl.BlockSpec((1, D_OUT), lambda i: (0, 0)),
        ],
        out_specs=pl.BlockSpec((BM, D_OUT), lambda i: (i, 0)),
        out_shape=jax.ShapeDtypeStruct((N, D_OUT), jnp.bfloat16),
    )(x, W1, b1.reshape(1, -1), W2, b2.reshape(1, -1))

    y, _ = pl.pallas_call(
        _mega_body,
        in_specs=[
            pl.BlockSpec(memory_space=pl.ANY),
            pl.BlockSpec(memory_space=pl.ANY),
            pl.BlockSpec(memory_space=pltpu.VMEM),
            pl.BlockSpec(memory_space=pltpu.SMEM),
        ],
        out_specs=[
            pl.BlockSpec(memory_space=pltpu.VMEM),
            pl.BlockSpec(memory_space=pl.ANY),
        ],
        out_shape=[
            jax.ShapeDtypeStruct((N, D_OUT), jnp.float32),
            jax.ShapeDtypeStruct((N, N), jnp.bfloat16),
        ],
        scratch_shapes=[
            pltpu.VMEM((N, N), jnp.bfloat16),
            pltpu.VMEM((8, N), jnp.bfloat16),
            pltpu.VMEM((2, _BR, N), jnp.float32),
            pltpu.VMEM((2, _BR, N), jnp.float32),
            pltpu.VMEM((2, _BC, N), jnp.bfloat16),
            pltpu.VMEM((2, N, D_OUT), jnp.bfloat16),
            pltpu.VMEM((2, N, D_OUT), jnp.bfloat16),
            pltpu.SemaphoreType.DMA((2,)),
            pltpu.SemaphoreType.DMA((2,)),
            pltpu.SemaphoreType.DMA((2,)),
            pltpu.SemaphoreType.DMA((2,)),
        ],
        compiler_params=pltpu.CompilerParams(
            vmem_limit_bytes=64 * 1024 * 1024),
    )(adj, poly_item, h, coefs)
    return y


# speedup vs baseline: 2.8261x; 1.0024x over previous
import math

import jax
import jax.numpy as jnp
from jax import lax
from jax.experimental import pallas as pl
from jax.experimental.pallas import tpu as pltpu

_K = 5
_BR = 64
_BC = 256
_NS = 4


def _mlp_body(x_ref, W1_ref, b1_ref, W2_ref, b2_ref, h_ref):
    h = jnp.dot(x_ref[...], W1_ref[...], preferred_element_type=jnp.float32)
    h = jnp.maximum(h + b1_ref[...], 0.0)
    h_ref[...] = (jnp.dot(h, W2_ref[...], preferred_element_type=jnp.float32)
                  + b2_ref[...]).astype(jnp.bfloat16)


def _mega_body(adj_hbm, poly_hbm, h_ref, c_ref,
               y_ref, p16_hbm,
               a16, fa, fp, pstage, pbuf, acc16,
               sa, sp, so, si):
    n = adj_hbm.shape[0]
    nb = n // _BR

    def a_in(b, slot):
        return pltpu.make_async_copy(
            adj_hbm.at[pl.ds(b * _BR, _BR), :], fa.at[slot], sa.at[slot])

    def p_in(b, slot):
        return pltpu.make_async_copy(
            poly_hbm.at[pl.ds(b * _BR, _BR), :], fp.at[slot], sp.at[slot])

    def p16_out(b, slot):
        return pltpu.make_async_copy(
            pstage.at[slot, pl.ds(0, _BR)],
            p16_hbm.at[pl.ds(b * _BR, _BR), :], so.at[slot])

    def p16_in(b, slot):
        return pltpu.make_async_copy(
            p16_hbm.at[pl.ds(b * _BC, _BC), :], pstage.at[slot], si.at[slot])

    h16 = h_ref[...]
    pbuf[0, :, :] = h16
    acc16[0, :, :] = (c_ref[_K, 0] * h16.astype(jnp.float32)).astype(jnp.bfloat16)

    for s in range(_NS):
        a_in(s, s).start()
        p_in(s, s).start()

    def body1(b, _):
        slot = lax.rem(b, _NS)
        rows = pl.ds(b * _BR, _BR)
        a_in(b, slot).wait()
        p_in(b, slot).wait()
        ablk = fa[slot].astype(jnp.bfloat16)
        pblk = fp[slot].astype(jnp.bfloat16)
        a16[rows, :] = ablk

        @pl.when(b >= _NS)
        def _wait_out():
            p16_out(b - _NS, slot).wait()

        pstage[slot, pl.ds(0, _BR), :] = pblk
        p16_out(b, slot).start()

        @pl.when(b + _NS < nb)
        def _next():
            a_in(b + _NS, slot).start()
            p_in(b + _NS, slot).start()

        pnew = jnp.dot(ablk, pbuf[0], preferred_element_type=jnp.float32)
        accn = (jnp.dot(pblk, acc16[0], preferred_element_type=jnp.float32)
                + c_ref[_K - 1, 0] * pnew)
        pbuf[1, rows, :] = pnew.astype(jnp.bfloat16)
        acc16[1, rows, :] = accn.astype(jnp.bfloat16)
        return 0

    lax.fori_loop(0, nb, body1, 0)
    for s in range(_NS):
        p16_out(nb - _NS + s, (nb - _NS + s) % _NS).wait()

    nbc = n // _BC
    total = (_K - 1) * nbc
    for f in range(_NS):
        p16_in(f % nbc, f % _NS).start()
    for f in range(total):
        t = 2 + f // nbc
        b = f % nbc
        cur = (t - 1) % 2
        nxt = t % 2
        last = t == _K
        slot = f % _NS
        rows = pl.ds(b * _BC, _BC)
        p16_in(b, slot).wait()
        pblk = pstage[slot, :, :]
        pnew = jnp.dot(a16[rows, :], pbuf[cur],
                       preferred_element_type=jnp.float32)
        accn = (jnp.dot(pblk, acc16[cur], preferred_element_type=jnp.float32)
                + c_ref[_K - t, 0] * pnew)
        if f + _NS < total:
            p16_in((f + _NS) % nbc, slot).start()
        if not last:
            pbuf[nxt, rows, :] = pnew.astype(jnp.bfloat16)
            acc16[nxt, rows, :] = accn.astype(jnp.bfloat16)
        else:
            m = jnp.max(accn, axis=1, keepdims=True)
            lse = (jnp.log(jnp.sum(jnp.exp(accn - m), axis=1,
                                   keepdims=True)) + m)
            y_ref[rows, :] = accn - lse


def kernel(x, adj, poly_item, W1, b1, W2, b2, filter_param):
    N, D_IN = x.shape
    D_HID = W1.shape[1]
    D_OUT = W2.shape[1]

    fp = jax.nn.relu(filter_param[:, 0])
    binom = jnp.asarray([math.comb(_K, i) / 2.0 ** _K for i in range(_K + 1)],
                        jnp.float32)
    coefs = jnp.zeros((8, 1), jnp.float32).at[:_K + 1, 0].set(binom * fp)

    BM = 256
    h = pl.pallas_call(
        _mlp_body,
        grid=(N // BM,),
        in_specs=[
            pl.BlockSpec((BM, D_IN), lambda i: (i, 0)),
            pl.BlockSpec((D_IN, D_HID), lambda i: (0, 0)),
            pl.BlockSpec((1, D_HID), lambda i: (0, 0)),
            pl.BlockSpec((D_HID, D_OUT), lambda i: (0, 0)),
            pl.BlockSpec((1, D_OUT), lambda i: (0, 0)),
        ],
        out_specs=pl.BlockSpec((BM, D_OUT), lambda i: (i, 0)),
        out_shape=jax.ShapeDtypeStruct((N, D_OUT), jnp.bfloat16),
    )(x, W1, b1.reshape(1, -1), W2, b2.reshape(1, -1))

    y, _ = pl.pallas_call(
        _mega_body,
        in_specs=[
            pl.BlockSpec(memory_space=pl.ANY),
            pl.BlockSpec(memory_space=pl.ANY),
            pl.BlockSpec(memory_space=pltpu.VMEM),
            pl.BlockSpec(memory_space=pltpu.SMEM),
        ],
        out_specs=[
            pl.BlockSpec(memory_space=pltpu.VMEM),
            pl.BlockSpec(memory_space=pl.ANY),
        ],
        out_shape=[
            jax.ShapeDtypeStruct((N, D_OUT), jnp.float32),
            jax.ShapeDtypeStruct((N, N), jnp.bfloat16),
        ],
        scratch_shapes=[
            pltpu.VMEM((N, N), jnp.bfloat16),
            pltpu.VMEM((_NS, _BR, N), jnp.float32),
            pltpu.VMEM((_NS, _BR, N), jnp.float32),
            pltpu.VMEM((_NS, _BC, N), jnp.bfloat16),
            pltpu.VMEM((2, N, D_OUT), jnp.bfloat16),
            pltpu.VMEM((2, N, D_OUT), jnp.bfloat16),
            pltpu.SemaphoreType.DMA((_NS,)),
            pltpu.SemaphoreType.DMA((_NS,)),
            pltpu.SemaphoreType.DMA((_NS,)),
            pltpu.SemaphoreType.DMA((_NS,)),
        ],
        compiler_params=pltpu.CompilerParams(
            vmem_limit_bytes=64 * 1024 * 1024),
    )(adj, poly_item, h, coefs)
    return y


# speedup vs baseline: 3.1864x; 1.1275x over previous
import math

import jax
import jax.numpy as jnp
from jax import lax
from jax.experimental import pallas as pl
from jax.experimental.pallas import tpu as pltpu

_K = 5
_BR = 128
_BC = 256
_NS = 4


def _mlp_body(x_ref, W1_ref, b1_ref, W2_ref, b2_ref, h_ref):
    h = jnp.dot(x_ref[...], W1_ref[...], preferred_element_type=jnp.float32)
    h = jnp.maximum(h + b1_ref[...], 0.0)
    h = (jnp.dot(h, W2_ref[...], preferred_element_type=jnp.float32)
         + b2_ref[...])
    h_ref[...] = h.T.astype(jnp.bfloat16)


def _dotT(vT, blk):
    return lax.dot_general(vT, blk, (((1,), (1,)), ((), ())),
                           preferred_element_type=jnp.float32)


def _mega_body(adj_hbm, poly_hbm, hT_ref, c_ref,
               y_ref,
               m16, fa, fp, pT, accT,
               sa, sp):
    n = adj_hbm.shape[0]
    nb = n // _BR
    nc = n // _BC

    def a_in(b, slot):
        return pltpu.make_async_copy(
            adj_hbm.at[pl.ds(b * _BR, _BR), :], fa.at[slot], sa.at[slot])

    def p_in(b, slot):
        return pltpu.make_async_copy(
            poly_hbm.at[pl.ds(b * _BR, _BR), :], fp.at[slot], sp.at[slot])

    hT = hT_ref[...]

    for s in range(_NS):
        a_in(s, s).start()

    def body_a(b, _):
        slot = lax.rem(b, _NS)
        rows = pl.ds(b * _BR, _BR)
        a_in(b, slot).wait()
        ablk = fa[slot].astype(jnp.bfloat16)
        m16[rows, :] = ablk

        @pl.when(b + _NS < nb)
        def _next():
            a_in(b + _NS, slot).start()

        pT[1, :, rows] = _dotT(hT, ablk).astype(jnp.bfloat16)
        return 0

    lax.fori_loop(0, nb, body_a, 0)

    for s in range(_NS):
        p_in(s, s).start()

    for t in range(2, _K + 1):
        for c in range(nc):
            chunk = pl.ds(c * _BC, _BC)
            pT[t, :, chunk] = _dotT(pT[t - 1], m16[chunk, :]).astype(
                jnp.bfloat16)

    acc0T = (c_ref[_K, 0] * hT.astype(jnp.float32)).astype(jnp.bfloat16)

    def body_p(b, _):
        slot = lax.rem(b, _NS)
        rows = pl.ds(b * _BR, _BR)
        p_in(b, slot).wait()
        pblk = fp[slot].astype(jnp.bfloat16)
        m16[rows, :] = pblk

        @pl.when(b + _NS < nb)
        def _next():
            p_in(b + _NS, slot).start()

        acc1T = (_dotT(acc0T, pblk)
                 + c_ref[_K - 1, 0] * pT[1, :, rows].astype(jnp.float32))
        accT[1, :, rows] = acc1T.astype(jnp.bfloat16)
        return 0

    lax.fori_loop(0, nb, body_p, 0)

    for t in range(2, _K + 1):
        cur = (t - 1) % 2
        nxt = t % 2
        last = t == _K
        for c in range(nc):
            chunk = pl.ds(c * _BC, _BC)
            accnT = (_dotT(accT[cur], m16[chunk, :])
                     + c_ref[_K - t, 0] * pT[t, :, chunk].astype(jnp.float32))
            if not last:
                accT[nxt, :, chunk] = accnT.astype(jnp.bfloat16)
            else:
                m = jnp.max(accnT, axis=0, keepdims=True)
                lse = (jnp.log(jnp.sum(jnp.exp(accnT - m), axis=0,
                                       keepdims=True)) + m)
                y_ref[chunk, :] = (accnT - lse).T


def kernel(x, adj, poly_item, W1, b1, W2, b2, filter_param):
    N, D_IN = x.shape
    D_HID = W1.shape[1]
    D_OUT = W2.shape[1]

    fp = jax.nn.relu(filter_param[:, 0])
    binom = jnp.asarray([math.comb(_K, i) / 2.0 ** _K for i in range(_K + 1)],
                        jnp.float32)
    coefs = jnp.zeros((8, 1), jnp.float32).at[:_K + 1, 0].set(binom * fp)

    BM = 256
    hT = pl.pallas_call(
        _mlp_body,
        grid=(N // BM,),
        in_specs=[
            pl.BlockSpec((BM, D_IN), lambda i: (i, 0)),
            pl.BlockSpec((D_IN, D_HID), lambda i: (0, 0)),
            pl.BlockSpec((1, D_HID), lambda i: (0, 0)),
            pl.BlockSpec((D_HID, D_OUT), lambda i: (0, 0)),
            pl.BlockSpec((1, D_OUT), lambda i: (0, 0)),
        ],
        out_specs=pl.BlockSpec((D_OUT, BM), lambda i: (0, i)),
        out_shape=jax.ShapeDtypeStruct((D_OUT, N), jnp.bfloat16),
    )(x, W1, b1.reshape(1, -1), W2, b2.reshape(1, -1))

    y = pl.pallas_call(
        _mega_body,
        in_specs=[
            pl.BlockSpec(memory_space=pl.ANY),
            pl.BlockSpec(memory_space=pl.ANY),
            pl.BlockSpec(memory_space=pltpu.VMEM),
            pl.BlockSpec(memory_space=pltpu.SMEM),
        ],
        out_specs=pl.BlockSpec(memory_space=pltpu.VMEM),
        out_shape=jax.ShapeDtypeStruct((N, D_OUT), jnp.float32),
        scratch_shapes=[
            pltpu.VMEM((N, N), jnp.bfloat16),
            pltpu.VMEM((_NS, _BR, N), jnp.float32),
            pltpu.VMEM((_NS, _BR, N), jnp.float32),
            pltpu.VMEM((_K + 1, D_OUT, N), jnp.bfloat16),
            pltpu.VMEM((2, D_OUT, N), jnp.bfloat16),
            pltpu.SemaphoreType.DMA((_NS,)),
            pltpu.SemaphoreType.DMA((_NS,)),
        ],
        compiler_params=pltpu.CompilerParams(
            vmem_limit_bytes=64 * 1024 * 1024),
    )(adj, poly_item, hT, coefs)
    return y


# speedup vs baseline: 3.1918x; 1.0017x over previous
import math

import jax
import jax.numpy as jnp
from jax import lax
from jax.experimental import pallas as pl
from jax.experimental.pallas import tpu as pltpu

_K = 5
_BR = 128
_BC = 1024
_NS = 4


def _mlp_body(x_ref, W1_ref, b1_ref, W2_ref, b2_ref, h_ref):
    h = jnp.dot(x_ref[...], W1_ref[...], preferred_element_type=jnp.float32)
    h = jnp.maximum(h + b1_ref[...], 0.0)
    h = (jnp.dot(h, W2_ref[...], preferred_element_type=jnp.float32)
         + b2_ref[...])
    h_ref[...] = h.T.astype(jnp.bfloat16)


def _dotT(vT, blk):
    return lax.dot_general(vT, blk, (((1,), (1,)), ((), ())),
                           preferred_element_type=jnp.float32)


def _mega_body(adj_hbm, poly_hbm, hT_ref, c_ref,
               y_ref,
               m16, fa, fp, pT, accT,
               sa, sp):
    n = adj_hbm.shape[0]
    nb = n // _BR
    nc = n // _BC

    def a_in(b, slot):
        return pltpu.make_async_copy(
            adj_hbm.at[pl.ds(b * _BR, _BR), :], fa.at[slot], sa.at[slot])

    def p_in(b, slot):
        return pltpu.make_async_copy(
            poly_hbm.at[pl.ds(b * _BR, _BR), :], fp.at[slot], sp.at[slot])

    hT = hT_ref[...]

    for s in range(_NS):
        a_in(s, s).start()

    def body_a(b, _):
        slot = lax.rem(b, _NS)
        rows = pl.ds(b * _BR, _BR)
        a_in(b, slot).wait()
        ablk = fa[slot].astype(jnp.bfloat16)
        m16[rows, :] = ablk

        @pl.when(b + _NS < nb)
        def _next():
            a_in(b + _NS, slot).start()

        pT[1, :, rows] = _dotT(hT, ablk).astype(jnp.bfloat16)
        return 0

    lax.fori_loop(0, nb, body_a, 0)

    for s in range(_NS):
        p_in(s, s).start()

    for t in range(2, _K + 1):
        for c in range(nc):
            chunk = pl.ds(c * _BC, _BC)
            pT[t, :, chunk] = _dotT(pT[t - 1], m16[chunk, :]).astype(
                jnp.bfloat16)

    acc0T = (c_ref[_K, 0] * hT.astype(jnp.float32)).astype(jnp.bfloat16)

    def body_p(b, _):
        slot = lax.rem(b, _NS)
        rows = pl.ds(b * _BR, _BR)
        p_in(b, slot).wait()
        pblk = fp[slot].astype(jnp.bfloat16)
        m16[rows, :] = pblk

        @pl.when(b + _NS < nb)
        def _next():
            p_in(b + _NS, slot).start()

        acc1T = (_dotT(acc0T, pblk)
                 + c_ref[_K - 1, 0] * pT[1, :, rows].astype(jnp.float32))
        accT[1, :, rows] = acc1T.astype(jnp.bfloat16)
        return 0

    lax.fori_loop(0, nb, body_p, 0)

    for t in range(2, _K + 1):
        cur = (t - 1) % 2
        nxt = t % 2
        last = t == _K
        for c in range(nc):
            chunk = pl.ds(c * _BC, _BC)
            accnT = (_dotT(accT[cur], m16[chunk, :])
                     + c_ref[_K - t, 0] * pT[t, :, chunk].astype(jnp.float32))
            if not last:
                accT[nxt, :, chunk] = accnT.astype(jnp.bfloat16)
            else:
                m = jnp.max(accnT, axis=0, keepdims=True)
                lse = (jnp.log(jnp.sum(jnp.exp(accnT - m), axis=0,
                                       keepdims=True)) + m)
                y_ref[chunk, :] = (accnT - lse).T


def kernel(x, adj, poly_item, W1, b1, W2, b2, filter_param):
    N, D_IN = x.shape
    D_HID = W1.shape[1]
    D_OUT = W2.shape[1]

    fp = jax.nn.relu(filter_param[:, 0])
    binom = jnp.asarray([math.comb(_K, i) / 2.0 ** _K for i in range(_K + 1)],
                        jnp.float32)
    coefs = jnp.zeros((8, 1), jnp.float32).at[:_K + 1, 0].set(binom * fp)

    BM = 256
    hT = pl.pallas_call(
        _mlp_body,
        grid=(N // BM,),
        in_specs=[
            pl.BlockSpec((BM, D_IN), lambda i: (i, 0)),
            pl.BlockSpec((D_IN, D_HID), lambda i: (0, 0)),
            pl.BlockSpec((1, D_HID), lambda i: (0, 0)),
            pl.BlockSpec((D_HID, D_OUT), lambda i: (0, 0)),
            pl.BlockSpec((1, D_OUT), lambda i: (0, 0)),
        ],
        out_specs=pl.BlockSpec((D_OUT, BM), lambda i: (0, i)),
        out_shape=jax.ShapeDtypeStruct((D_OUT, N), jnp.bfloat16),
    )(x, W1, b1.reshape(1, -1), W2, b2.reshape(1, -1))

    y = pl.pallas_call(
        _mega_body,
        in_specs=[
            pl.BlockSpec(memory_space=pl.ANY),
            pl.BlockSpec(memory_space=pl.ANY),
            pl.BlockSpec(memory_space=pltpu.VMEM),
            pl.BlockSpec(memory_space=pltpu.SMEM),
        ],
        out_specs=pl.BlockSpec(memory_space=pltpu.VMEM),
        out_shape=jax.ShapeDtypeStruct((N, D_OUT), jnp.float32),
        scratch_shapes=[
            pltpu.VMEM((N, N), jnp.bfloat16),
            pltpu.VMEM((_NS, _BR, N), jnp.float32),
            pltpu.VMEM((_NS, _BR, N), jnp.float32),
            pltpu.VMEM((_K + 1, D_OUT, N), jnp.bfloat16),
            pltpu.VMEM((2, D_OUT, N), jnp.bfloat16),
            pltpu.SemaphoreType.DMA((_NS,)),
            pltpu.SemaphoreType.DMA((_NS,)),
        ],
        compiler_params=pltpu.CompilerParams(
            vmem_limit_bytes=64 * 1024 * 1024),
    )(adj, poly_item, hT, coefs)
    return y
